# baseline (speedup 1.0000x reference)
import functools
import jax
import jax.numpy as jnp
from jax.experimental import pallas as pl
from jax.experimental.pallas import tpu as pltpu

_PROCESS_NOISE = 0.1
_OBS_NOISE = 0.5
_P = 32768
_NB = 256
_BLK = 128


def _seq_scan_sublanes(x):
    rows = [x[0:1, :]]
    for k in range(1, x.shape[0]):
        rows.append(rows[-1] + x[k:k + 1, :])
    return jnp.concatenate(rows, axis=0)


def _stage_a_kernel(predT_ref, lw_ref, obs_ref, C_ref, u0_ref, n_ref):
    predT = predT_ref[0]
    obsT = jnp.dot(C_ref[...], predT, preferred_element_type=jnp.float32)
    d0 = obs_ref[0, 0, 0] - obsT[0:1, :]
    d1 = obs_ref[0, 0, 1] - obsT[1:2, :]
    ll = -0.5 * (d0 * d0 + d1 * d1) / (_OBS_NOISE ** 2)
    nlw = lw_ref[0] + ll

    m = jnp.max(nlw)
    m = jnp.where(jnp.isfinite(m), m, 0.0)
    ex = jnp.exp(nlw - m)
    acc = ex[:, 0:_BLK]
    for c in range(1, _NB):
        acc = acc + ex[:, c * _BLK:(c + 1) * _BLK]
    g = acc[:, 0:8]
    for gi in range(1, 16):
        g = g + acc[:, gi * 8:(gi + 1) * 8]
    g = g[:, 0:4] + g[:, 4:8]
    g = g[:, 0:2] + g[:, 2:4]
    s = g[:, 0:1] + g[:, 1:2]
    lse = jnp.log(s[0, 0]) + m
    w = jnp.exp(nlw - lse)

    w_mat = jnp.concatenate(
        [w[:, c * _BLK:(c + 1) * _BLK] for c in range(_NB)], axis=0)
    wT = w_mat.T
    inner1 = _seq_scan_sublanes(wT)
    totals = inner1[127:128, :]
    t2 = jnp.concatenate([totals[:, 0:128], totals[:, 128:256]], axis=0)
    t2T = t2.T
    incl2 = _seq_scan_sublanes(t2T)
    full2c1 = incl2[:, 1:2] + incl2[127:128, 0:1]
    full2 = jnp.concatenate([incl2[:, 0:1], full2c1], axis=1)
    f2T = full2.T
    flat2 = jnp.concatenate([f2T[0:1, :], f2T[1:2, :]], axis=1)
    off2 = jnp.concatenate(
        [jnp.zeros((1, 1), jnp.float32), flat2[:, 0:255]], axis=1)
    csT = inner1 + off2

    u0 = u0_ref[0, 0, 0]
    t = csT * jnp.float32(_P) - u0
    f = jnp.floor(t)
    ind = jnp.zeros_like(csT)
    for dlt in (-1.0, 0.0, 1.0):
        ug = ((f + dlt) + u0) / jnp.float32(_P)
        ind = ind + jnp.where(ug <= csT, 1.0, 0.0)
    n_val = f.astype(jnp.int32) - 1 + ind.astype(jnp.int32)
    n_val = jnp.clip(n_val, 0, _P)
    n_ref[0] = n_val.T


@functools.partial(jax.jit, static_argnums=())
def _stage_a(predT, lw, obs, C, u0):
    B = predT.shape[0]
    grid = (B,)
    return pl.pallas_call(
        _stage_a_kernel,
        grid=grid,
        in_specs=[
            pl.BlockSpec((1, 4, _P), lambda b: (b, 0, 0)),
            pl.BlockSpec((1, 1, _P), lambda b: (b, 0, 0)),
            pl.BlockSpec((1, 1, 2), lambda b: (b, 0, 0), memory_space=pltpu.SMEM),
            pl.BlockSpec((2, 4), lambda b: (0, 0)),
            pl.BlockSpec((1, 1, 1), lambda b: (b, 0, 0), memory_space=pltpu.SMEM),
        ],
        out_specs=pl.BlockSpec((1, _NB, _BLK), lambda b: (b, 0, 0)),
        out_shape=jax.ShapeDtypeStruct((B, _NB, _BLK), jnp.int32),
    )(predT, lw, obs, C, u0)


def kernel(particles, log_weights, observation, A, C):
    B_, P_, D = particles.shape
    key = jax.random.key(42)
    kn, ku = jax.random.split(key)
    noise = jax.random.normal(kn, particles.shape, dtype=particles.dtype)
    u0 = jax.random.uniform(ku, (B_, 1), dtype=particles.dtype)

    pred = particles @ A.T + _PROCESS_NOISE * noise
    predT = jnp.transpose(pred, (0, 2, 1))
    n3 = _stage_a(predT, log_weights.reshape(B_, 1, P_),
                  observation.reshape(B_, 1, 2), C, u0.reshape(B_, 1, 1))
    n_arr = n3.reshape(B_, P_)

    iota = jnp.arange(P_, dtype=jnp.int32)
    indices = jax.vmap(lambda nv: jnp.searchsorted(nv, iota, side='right'))(n_arr)
    indices = jnp.clip(indices, 0, P_ - 1)
    resampled = jnp.take_along_axis(pred, indices[:, :, None], axis=1)
    uniform_log_w = jnp.full((B_, P_), -jnp.log(float(P_)), dtype=particles.dtype)
    return (resampled, uniform_log_w)
